# baseline (speedup 1.0000x reference)
import functools

import jax
import jax.numpy as jnp
from jax import lax
from jax.experimental import pallas as pl
from jax.experimental.pallas import tpu as pltpu
from jax.experimental.pallas import tpu_sc as plsc

N = 10000
D = 128
NPAD = 10240
NC = 2
NS = 16
NW = NC * NS
CH = 64
NBUF = 4
PIECE = 40
C0 = 320
C1 = 0
TOTCH = NS * (C0 + C1)
EPAD = TOTCH * CH
RPW = NPAD // NS
PADIDX = NPAD - 1

_mesh = plsc.VectorSubcoreMesh(core_axis_name="c", subcore_axis_name="s")


@functools.partial(
    pl.kernel,
    out_type=jax.ShapeDtypeStruct((NC, NPAD, 16), jnp.float32),
    mesh=_mesh,
    scratch_types=[
        pltpu.VMEM((TOTCH // NW, CH), jnp.int32),
        pltpu.VMEM((CH, 16), jnp.float32),
        pltpu.VMEM_SHARED((NPAD, 16), jnp.float32),
    ],
)
def _deg_sc(dst_hbm, zo_hbm, out_hbm, dst_v, buf_v, deg_sh):
    c = lax.axis_index("c")
    s = lax.axis_index("s")
    w = c * NS + s
    pltpu.sync_copy(zo_hbm.at[0], buf_v)
    for k in range(RPW // CH):
        pltpu.sync_copy(buf_v, deg_sh.at[pl.ds(s * RPW + k * CH, CH)])
    plsc.subcore_barrier()
    pltpu.sync_copy(dst_hbm.at[pl.ds(w * (TOTCH // NW), TOTCH // NW)], dst_v)
    pltpu.sync_copy(zo_hbm.at[1], buf_v)

    def body(i, carry):
        pltpu.sync_copy(buf_v, deg_sh.at[dst_v.at[i]], add=True)
        return carry

    lax.fori_loop(0, TOTCH // NW, body, 0)
    plsc.subcore_barrier()
    pltpu.sync_copy(deg_sh.at[pl.ds(s * RPW, RPW)],
                    out_hbm.at[c, pl.ds(s * RPW, RPW)])


@functools.partial(
    pl.kernel,
    out_type=jax.ShapeDtypeStruct((NC, NPAD, D), jnp.float32),
    mesh=_mesh,
    scratch_types=(
        [pltpu.VMEM((PIECE, CH), jnp.int32)] * 2
        + [pltpu.VMEM((CH, D), jnp.float32)] * NBUF
        + [pltpu.VMEM_SHARED((NPAD, D), jnp.float32)]
        + [pltpu.SemaphoreType.DMA] * (2 * NBUF)
    ),
)
def _agg_sc(y_hbm, src_hbm, dst_hbm, out_hbm, src_v, dst_v, *rest):
    bufs = rest[:NBUF]
    acc_sh = rest[NBUF]
    gs = rest[NBUF + 1:2 * NBUF + 1]
    ss = rest[2 * NBUF + 1:]
    c = lax.axis_index("c")
    s = lax.axis_index("s")
    w = c * NS + s

    def gather(i, b):
        return pltpu.async_copy(y_hbm.at[src_v.at[i]], bufs[b], gs[b])

    def gwait(i, b):
        pltpu.make_async_copy(y_hbm.at[src_v.at[i]], bufs[b], gs[b]).wait()

    def scatter(i, b):
        return pltpu.async_copy(bufs[b], acc_sh.at[dst_v.at[i]], ss[b],
                                add=True)

    def swait(i, b):
        pltpu.make_async_copy(bufs[b], acc_sh.at[dst_v.at[i]], ss[b]).wait()

    pltpu.sync_copy(y_hbm.at[pl.ds(NPAD - CH, CH)], bufs[0])
    for k in range(RPW // CH):
        pltpu.sync_copy(bufs[0], acc_sh.at[pl.ds(s * RPW + k * CH, CH)])
    plsc.subcore_barrier()

    cbase = jnp.where(c == 0, s * C0, NS * C0 + s * C1)
    npieces = jnp.where(c == 0, C0 // PIECE, C1 // PIECE)

    def piece(q, carry):
        pbase = cbase + q * PIECE
        pltpu.sync_copy(src_hbm.at[pl.ds(pbase, PIECE)], src_v)
        pltpu.sync_copy(dst_hbm.at[pl.ds(pbase, PIECE)], dst_v)
        for b in range(NBUF):
            gather(b, b)

        def body(qq, carry2):
            i = NBUF * qq
            for b in range(NBUF):
                gwait(i + b, b)
                scatter(i + b, b)
            for b in range(NBUF):
                swait(i + b, b)
                gather(i + NBUF + b, b)
            return carry2

        lax.fori_loop(0, PIECE // NBUF - 1, body, 0)
        i = PIECE - NBUF
        for b in range(NBUF):
            gwait(i + b, b)
            scatter(i + b, b)
        for b in range(NBUF):
            swait(i + b, b)
        return carry

    lax.fori_loop(0, npieces, piece, 0)
    plsc.subcore_barrier()
    pltpu.sync_copy(acc_sh.at[pl.ds(s * RPW, RPW)],
                    out_hbm.at[c, pl.ds(s * RPW, RPW)])


_R = 1024
_G = NPAD // _R


def _dinv_blk(dp_ref):
    deg = dp_ref[0, :, 0:1] + dp_ref[1, :, 0:1] + 1.0
    return lax.rsqrt(deg)


def _pre_tc(x_ref, w_ref, dp_ref, y_ref):
    dinv = _dinv_blk(dp_ref)
    y_ref[...] = jnp.dot(x_ref[...] * dinv, w_ref[...],
                         preferred_element_type=jnp.float32)


def _mid_tc(p_ref, y1_ref, dp_ref, b_ref, w_ref, o_ref):
    dinv = _dinv_blk(dp_ref)
    a = p_ref[0] + p_ref[1] + y1_ref[...]
    h = jnp.maximum(a * dinv + b_ref[...], 0.0)
    y2 = jnp.dot(h * dinv, w_ref[...], preferred_element_type=jnp.float32)
    row = pl.program_id(0) * _R + lax.broadcasted_iota(jnp.int32, (_R, 1), 0)
    o_ref[...] = jnp.where(row < N, y2, 0.0)


def _post_tc(p_ref, y2_ref, dp_ref, b_ref, x_ref, o_ref):
    dinv = _dinv_blk(dp_ref)
    a = p_ref[0] + p_ref[1] + y2_ref[...]
    h = jnp.maximum(a * dinv + b_ref[...], 0.0)
    o_ref[...] = (x_ref[...] + h) * 0.5


_rowspec = pl.BlockSpec((_R, D), lambda i: (i, 0))
_wspec = pl.BlockSpec((D, D), lambda i: (0, 0))
_dpspec = pl.BlockSpec((NC, _R, 16), lambda i: (0, i, 0))
_pspec = pl.BlockSpec((NC, _R, D), lambda i: (0, i, 0))
_bspec = pl.BlockSpec((1, D), lambda i: (0, 0))
_out128 = jax.ShapeDtypeStruct((NPAD, D), jnp.float32)

_pre_call = pl.pallas_call(
    _pre_tc, grid=(_G,),
    in_specs=[_rowspec, _wspec, _dpspec],
    out_specs=_rowspec, out_shape=_out128)

_mid_call = pl.pallas_call(
    _mid_tc, grid=(_G,),
    in_specs=[_pspec, _rowspec, _dpspec, _bspec, _wspec],
    out_specs=_rowspec, out_shape=_out128)

_post_call = pl.pallas_call(
    _post_tc, grid=(_G,),
    in_specs=[_pspec, _rowspec, _dpspec, _bspec, _rowspec],
    out_specs=_rowspec, out_shape=_out128)


def kernel(inputs, edge_index, W1, b1, W2, b2):
    src = edge_index[0].astype(jnp.int32)
    dst = edge_index[1].astype(jnp.int32)
    e = src.shape[0]
    pad = jnp.full((EPAD - e,), PADIDX, jnp.int32)
    src_r = jnp.concatenate([src, pad]).reshape(TOTCH, CH)
    dst_r = jnp.concatenate([dst, pad]).reshape(TOTCH, CH)
    x_p = jnp.concatenate(
        [inputs.astype(jnp.float32), jnp.zeros((NPAD - N, D), jnp.float32)])
    zo = jnp.concatenate([jnp.zeros((1, CH, 16), jnp.float32),
                          jnp.ones((1, CH, 16), jnp.float32)])
    b1r = b1.astype(jnp.float32).reshape(1, D)
    b2r = b2.astype(jnp.float32).reshape(1, D)

    deg_parts = _deg_sc(dst_r, zo)
    y1 = _pre_call(x_p, W1.astype(jnp.float32), deg_parts)
    p1 = _agg_sc(y1, src_r, dst_r)
    y2 = _mid_call(p1, y1, deg_parts, b1r, W2.astype(jnp.float32))
    p2 = _agg_sc(y2, src_r, dst_r)
    out_p = _post_call(p2, y2, deg_parts, b2r, x_p)
    return out_p[:N]


# speedup vs baseline: 1.2977x; 1.2977x over previous
import functools

import jax
import jax.numpy as jnp
from jax import lax
from jax.experimental import pallas as pl
from jax.experimental.pallas import tpu as pltpu
from jax.experimental.pallas import tpu_sc as plsc

N = 10000
D = 128
NPAD = 10240
NC = 2
NS = 16
NW = NC * NS
CH = 64
NBUF = 4
PIECE = 40
C0 = 280
C1 = 40
TOTCH = NS * (C0 + C1)
EPAD = TOTCH * CH
RPW = NPAD // NS
PADIDX = NPAD - 1

_mesh = plsc.VectorSubcoreMesh(core_axis_name="c", subcore_axis_name="s")


@functools.partial(
    pl.kernel,
    out_type=jax.ShapeDtypeStruct((NC, NPAD, 16), jnp.float32),
    mesh=_mesh,
    scratch_types=[
        pltpu.VMEM((TOTCH // NW, CH), jnp.int32),
        pltpu.VMEM((CH, 16), jnp.float32),
        pltpu.VMEM_SHARED((NPAD, 16), jnp.float32),
    ],
)
def _deg_sc(dst_hbm, zo_hbm, out_hbm, dst_v, buf_v, deg_sh):
    c = lax.axis_index("c")
    s = lax.axis_index("s")
    w = c * NS + s
    pltpu.sync_copy(zo_hbm.at[0], buf_v)
    for k in range(RPW // CH):
        pltpu.sync_copy(buf_v, deg_sh.at[pl.ds(s * RPW + k * CH, CH)])
    plsc.subcore_barrier()
    pltpu.sync_copy(dst_hbm.at[pl.ds(w * (TOTCH // NW), TOTCH // NW)], dst_v)
    pltpu.sync_copy(zo_hbm.at[1], buf_v)

    def body(i, carry):
        pltpu.sync_copy(buf_v, deg_sh.at[dst_v.at[i]], add=True)
        return carry

    lax.fori_loop(0, TOTCH // NW, body, 0)
    plsc.subcore_barrier()
    pltpu.sync_copy(deg_sh.at[pl.ds(s * RPW, RPW)],
                    out_hbm.at[c, pl.ds(s * RPW, RPW)])


@functools.partial(
    pl.kernel,
    out_type=jax.ShapeDtypeStruct((NC, NPAD, D), jnp.float32),
    mesh=_mesh,
    scratch_types=(
        [pltpu.VMEM((PIECE, CH), jnp.int32)] * 2
        + [pltpu.VMEM((CH, D), jnp.float32)] * NBUF
        + [pltpu.VMEM_SHARED((NPAD, D), jnp.float32)]
        + [pltpu.SemaphoreType.DMA] * (2 * NBUF)
    ),
)
def _agg_sc(y_hbm, src_hbm, dst_hbm, out_hbm, src_v, dst_v, *rest):
    bufs = rest[:NBUF]
    acc_sh = rest[NBUF]
    gs = rest[NBUF + 1:2 * NBUF + 1]
    ss = rest[2 * NBUF + 1:]
    c = lax.axis_index("c")
    s = lax.axis_index("s")
    w = c * NS + s

    def gather(i, b):
        return pltpu.async_copy(y_hbm.at[src_v.at[i]], bufs[b], gs[b])

    def gwait(i, b):
        pltpu.make_async_copy(y_hbm.at[src_v.at[i]], bufs[b], gs[b]).wait()

    def scatter(i, b):
        return pltpu.async_copy(bufs[b], acc_sh.at[dst_v.at[i]], ss[b],
                                add=True)

    def swait(i, b):
        pltpu.make_async_copy(bufs[b], acc_sh.at[dst_v.at[i]], ss[b]).wait()

    pltpu.sync_copy(y_hbm.at[pl.ds(NPAD - CH, CH)], bufs[0])
    for k in range(RPW // CH):
        pltpu.sync_copy(bufs[0], acc_sh.at[pl.ds(s * RPW + k * CH, CH)])
    plsc.subcore_barrier()

    cbase = jnp.where(c == 0, s * C0, NS * C0 + s * C1)
    npieces = jnp.where(c == 0, C0 // PIECE, C1 // PIECE)

    def piece(q, carry):
        pbase = cbase + q * PIECE
        pltpu.sync_copy(src_hbm.at[pl.ds(pbase, PIECE)], src_v)
        pltpu.sync_copy(dst_hbm.at[pl.ds(pbase, PIECE)], dst_v)
        for b in range(NBUF):
            gather(b, b)

        def body(qq, carry2):
            i = NBUF * qq
            for b in range(NBUF):
                gwait(i + b, b)
                scatter(i + b, b)
            for b in range(NBUF):
                swait(i + b, b)
                gather(i + NBUF + b, b)
            return carry2

        lax.fori_loop(0, PIECE // NBUF - 1, body, 0)
        i = PIECE - NBUF
        for b in range(NBUF):
            gwait(i + b, b)
            scatter(i + b, b)
        for b in range(NBUF):
            swait(i + b, b)
        return carry

    lax.fori_loop(0, npieces, piece, 0)
    plsc.subcore_barrier()
    pltpu.sync_copy(acc_sh.at[pl.ds(s * RPW, RPW)],
                    out_hbm.at[c, pl.ds(s * RPW, RPW)])


_R = 1024
_G = NPAD // _R


def _dinv_blk(dp_ref):
    deg = dp_ref[0, :, 0:1] + dp_ref[1, :, 0:1] + 1.0
    return lax.rsqrt(deg)


def _pre_tc(x_ref, w_ref, dp_ref, y_ref):
    dinv = _dinv_blk(dp_ref)
    y_ref[...] = jnp.dot(x_ref[...] * dinv, w_ref[...],
                         preferred_element_type=jnp.float32)


def _mid_tc(p_ref, y1_ref, dp_ref, b_ref, w_ref, o_ref):
    dinv = _dinv_blk(dp_ref)
    a = p_ref[0] + p_ref[1] + y1_ref[...]
    h = jnp.maximum(a * dinv + b_ref[...], 0.0)
    y2 = jnp.dot(h * dinv, w_ref[...], preferred_element_type=jnp.float32)
    row = pl.program_id(0) * _R + lax.broadcasted_iota(jnp.int32, (_R, 1), 0)
    o_ref[...] = jnp.where(row < N, y2, 0.0)


def _post_tc(p_ref, y2_ref, dp_ref, b_ref, x_ref, o_ref):
    dinv = _dinv_blk(dp_ref)
    a = p_ref[0] + p_ref[1] + y2_ref[...]
    h = jnp.maximum(a * dinv + b_ref[...], 0.0)
    o_ref[...] = (x_ref[...] + h) * 0.5


_rowspec = pl.BlockSpec((_R, D), lambda i: (i, 0))
_wspec = pl.BlockSpec((D, D), lambda i: (0, 0))
_dpspec = pl.BlockSpec((NC, _R, 16), lambda i: (0, i, 0))
_pspec = pl.BlockSpec((NC, _R, D), lambda i: (0, i, 0))
_bspec = pl.BlockSpec((1, D), lambda i: (0, 0))
_out128 = jax.ShapeDtypeStruct((NPAD, D), jnp.float32)

_pre_call = pl.pallas_call(
    _pre_tc, grid=(_G,),
    in_specs=[_rowspec, _wspec, _dpspec],
    out_specs=_rowspec, out_shape=_out128)

_mid_call = pl.pallas_call(
    _mid_tc, grid=(_G,),
    in_specs=[_pspec, _rowspec, _dpspec, _bspec, _wspec],
    out_specs=_rowspec, out_shape=_out128)

_post_call = pl.pallas_call(
    _post_tc, grid=(_G,),
    in_specs=[_pspec, _rowspec, _dpspec, _bspec, _rowspec],
    out_specs=_rowspec, out_shape=_out128)


def kernel(inputs, edge_index, W1, b1, W2, b2):
    src = edge_index[0].astype(jnp.int32)
    dst = edge_index[1].astype(jnp.int32)
    e = src.shape[0]
    pad = jnp.full((EPAD - e,), PADIDX, jnp.int32)
    src_r = jnp.concatenate([src, pad]).reshape(TOTCH, CH)
    dst_r = jnp.concatenate([dst, pad]).reshape(TOTCH, CH)
    x_p = jnp.concatenate(
        [inputs.astype(jnp.float32), jnp.zeros((NPAD - N, D), jnp.float32)])
    zo = jnp.concatenate([jnp.zeros((1, CH, 16), jnp.float32),
                          jnp.ones((1, CH, 16), jnp.float32)])
    b1r = b1.astype(jnp.float32).reshape(1, D)
    b2r = b2.astype(jnp.float32).reshape(1, D)

    deg_parts = _deg_sc(dst_r, zo)
    y1 = _pre_call(x_p, W1.astype(jnp.float32), deg_parts)
    p1 = _agg_sc(y1, src_r, dst_r)
    y2 = _mid_call(p1, y1, deg_parts, b1r, W2.astype(jnp.float32))
    p2 = _agg_sc(y2, src_r, dst_r)
    out_p = _post_call(p2, y2, deg_parts, b2r, x_p)
    return out_p[:N]


# speedup vs baseline: 1.3045x; 1.0052x over previous
import functools

import jax
import jax.numpy as jnp
from jax import lax
from jax.experimental import pallas as pl
from jax.experimental.pallas import tpu as pltpu
from jax.experimental.pallas import tpu_sc as plsc

N = 10000
D = 128
NPAD = 10240
NC = 2
NS = 16
NW = NC * NS
CH = 64
NBUF = 4
PIECE = 40
C0 = 280
C1 = 40
TOTCH = NS * (C0 + C1)
EPAD = TOTCH * CH
RPW = NPAD // NS
PADIDX = NPAD - 1

_mesh = plsc.VectorSubcoreMesh(core_axis_name="c", subcore_axis_name="s")


@functools.partial(
    pl.kernel,
    out_type=jax.ShapeDtypeStruct((NC, NPAD, 16), jnp.float32),
    mesh=_mesh,
    scratch_types=[
        pltpu.VMEM((TOTCH // NW, CH), jnp.int32),
        pltpu.VMEM((CH, 16), jnp.float32),
        pltpu.VMEM_SHARED((NPAD, 16), jnp.float32),
        pltpu.SemaphoreType.DMA,
        pltpu.SemaphoreType.DMA,
        pltpu.SemaphoreType.DMA,
        pltpu.SemaphoreType.DMA,
    ],
)
def _deg_sc(dst_hbm, zo_hbm, out_hbm, dst_v, buf_v, deg_sh,
            ds0, ds1, ds2, ds3):
    c = lax.axis_index("c")
    s = lax.axis_index("s")
    w = c * NS + s
    pltpu.sync_copy(zo_hbm.at[0], buf_v)
    for k in range(RPW // CH):
        pltpu.sync_copy(buf_v, deg_sh.at[pl.ds(s * RPW + k * CH, CH)])
    plsc.subcore_barrier()
    pltpu.sync_copy(dst_hbm.at[pl.ds(w * (TOTCH // NW), TOTCH // NW)], dst_v)
    pltpu.sync_copy(zo_hbm.at[1], buf_v)

    dsems = (ds0, ds1, ds2, ds3)

    def body(q, carry):
        i = 4 * q
        for b in range(4):
            pltpu.async_copy(buf_v, deg_sh.at[dst_v.at[i + b]], dsems[b],
                             add=True)
        for b in range(4):
            pltpu.make_async_copy(buf_v, deg_sh.at[dst_v.at[i + b]],
                                  dsems[b]).wait()
        return carry

    lax.fori_loop(0, TOTCH // NW // 4, body, 0)
    plsc.subcore_barrier()
    pltpu.sync_copy(deg_sh.at[pl.ds(s * RPW, RPW)],
                    out_hbm.at[c, pl.ds(s * RPW, RPW)])


@functools.partial(
    pl.kernel,
    out_type=jax.ShapeDtypeStruct((NC, NPAD, D), jnp.float32),
    mesh=_mesh,
    scratch_types=(
        [pltpu.VMEM((PIECE, CH), jnp.int32)] * 2
        + [pltpu.VMEM((CH, D), jnp.float32)] * NBUF
        + [pltpu.VMEM_SHARED((NPAD, D), jnp.float32)]
        + [pltpu.SemaphoreType.DMA] * (2 * NBUF)
    ),
)
def _agg_sc(y_hbm, src_hbm, dst_hbm, out_hbm, src_v, dst_v, *rest):
    bufs = rest[:NBUF]
    acc_sh = rest[NBUF]
    gs = rest[NBUF + 1:2 * NBUF + 1]
    ss = rest[2 * NBUF + 1:]
    c = lax.axis_index("c")
    s = lax.axis_index("s")
    w = c * NS + s

    def gather(i, b):
        return pltpu.async_copy(y_hbm.at[src_v.at[i]], bufs[b], gs[b])

    def gwait(i, b):
        pltpu.make_async_copy(y_hbm.at[src_v.at[i]], bufs[b], gs[b]).wait()

    def scatter(i, b):
        return pltpu.async_copy(bufs[b], acc_sh.at[dst_v.at[i]], ss[b],
                                add=True)

    def swait(i, b):
        pltpu.make_async_copy(bufs[b], acc_sh.at[dst_v.at[i]], ss[b]).wait()

    pltpu.sync_copy(y_hbm.at[pl.ds(NPAD - CH, CH)], bufs[0])
    for k in range(RPW // CH):
        pltpu.sync_copy(bufs[0], acc_sh.at[pl.ds(s * RPW + k * CH, CH)])
    plsc.subcore_barrier()

    cbase = jnp.where(c == 0, s * C0, NS * C0 + s * C1)
    npieces = jnp.where(c == 0, C0 // PIECE, C1 // PIECE)

    def piece(q, carry):
        pbase = cbase + q * PIECE
        pltpu.sync_copy(src_hbm.at[pl.ds(pbase, PIECE)], src_v)
        pltpu.sync_copy(dst_hbm.at[pl.ds(pbase, PIECE)], dst_v)
        for b in range(NBUF):
            gather(b, b)

        def body(qq, carry2):
            i = NBUF * qq
            for b in range(NBUF):
                gwait(i + b, b)
                scatter(i + b, b)
            for b in range(NBUF):
                swait(i + b, b)
                gather(i + NBUF + b, b)
            return carry2

        lax.fori_loop(0, PIECE // NBUF - 1, body, 0)
        i = PIECE - NBUF
        for b in range(NBUF):
            gwait(i + b, b)
            scatter(i + b, b)
        for b in range(NBUF):
            swait(i + b, b)
        return carry

    lax.fori_loop(0, npieces, piece, 0)
    plsc.subcore_barrier()
    pltpu.sync_copy(acc_sh.at[pl.ds(s * RPW, RPW)],
                    out_hbm.at[c, pl.ds(s * RPW, RPW)])


_R = 1024
_G = NPAD // _R


def _dinv_blk(dp_ref):
    deg = dp_ref[0, :, 0:1] + dp_ref[1, :, 0:1] + 1.0
    return lax.rsqrt(deg)


def _pre_tc(x_ref, w_ref, dp_ref, y_ref):
    dinv = _dinv_blk(dp_ref)
    y_ref[...] = jnp.dot(x_ref[...] * dinv, w_ref[...],
                         preferred_element_type=jnp.float32)


def _mid_tc(p_ref, y1_ref, dp_ref, b_ref, w_ref, o_ref):
    dinv = _dinv_blk(dp_ref)
    a = p_ref[0] + p_ref[1] + y1_ref[...]
    h = jnp.maximum(a * dinv + b_ref[...], 0.0)
    y2 = jnp.dot(h * dinv, w_ref[...], preferred_element_type=jnp.float32)
    row = pl.program_id(0) * _R + lax.broadcasted_iota(jnp.int32, (_R, 1), 0)
    o_ref[...] = jnp.where(row < N, y2, 0.0)


def _post_tc(p_ref, y2_ref, dp_ref, b_ref, x_ref, o_ref):
    dinv = _dinv_blk(dp_ref)
    a = p_ref[0] + p_ref[1] + y2_ref[...]
    h = jnp.maximum(a * dinv + b_ref[...], 0.0)
    o_ref[...] = (x_ref[...] + h) * 0.5


_rowspec = pl.BlockSpec((_R, D), lambda i: (i, 0))
_wspec = pl.BlockSpec((D, D), lambda i: (0, 0))
_dpspec = pl.BlockSpec((NC, _R, 16), lambda i: (0, i, 0))
_pspec = pl.BlockSpec((NC, _R, D), lambda i: (0, i, 0))
_bspec = pl.BlockSpec((1, D), lambda i: (0, 0))
_out128 = jax.ShapeDtypeStruct((NPAD, D), jnp.float32)

_pre_call = pl.pallas_call(
    _pre_tc, grid=(_G,),
    in_specs=[_rowspec, _wspec, _dpspec],
    out_specs=_rowspec, out_shape=_out128)

_mid_call = pl.pallas_call(
    _mid_tc, grid=(_G,),
    in_specs=[_pspec, _rowspec, _dpspec, _bspec, _wspec],
    out_specs=_rowspec, out_shape=_out128)

_post_call = pl.pallas_call(
    _post_tc, grid=(_G,),
    in_specs=[_pspec, _rowspec, _dpspec, _bspec, _rowspec],
    out_specs=_rowspec, out_shape=_out128)


def kernel(inputs, edge_index, W1, b1, W2, b2):
    src = edge_index[0].astype(jnp.int32)
    dst = edge_index[1].astype(jnp.int32)
    e = src.shape[0]
    pad = jnp.full((EPAD - e,), PADIDX, jnp.int32)
    src_r = jnp.concatenate([src, pad]).reshape(TOTCH, CH)
    dst_r = jnp.concatenate([dst, pad]).reshape(TOTCH, CH)
    x_p = jnp.concatenate(
        [inputs.astype(jnp.float32), jnp.zeros((NPAD - N, D), jnp.float32)])
    zo = jnp.concatenate([jnp.zeros((1, CH, 16), jnp.float32),
                          jnp.ones((1, CH, 16), jnp.float32)])
    b1r = b1.astype(jnp.float32).reshape(1, D)
    b2r = b2.astype(jnp.float32).reshape(1, D)

    deg_parts = _deg_sc(dst_r, zo)
    y1 = _pre_call(x_p, W1.astype(jnp.float32), deg_parts)
    p1 = _agg_sc(y1, src_r, dst_r)
    y2 = _mid_call(p1, y1, deg_parts, b1r, W2.astype(jnp.float32))
    p2 = _agg_sc(y2, src_r, dst_r)
    out_p = _post_call(p2, y2, deg_parts, b2r, x_p)
    return out_p[:N]
